# baseline (speedup 1.0000x reference)
import functools
import jax
import jax.numpy as jnp
from jax import lax
from jax.experimental import pallas as pl
from jax.experimental.pallas import tpu as pltpu
from jax.experimental.pallas import tpu_sc as plsc

NU = 25000
NI = 25000
N = NU + NI
NPAD = 50048
E = 800000
EPAD = 819200
D = 64
DH = 32
B = 4096
NTILES = 16
CH = 128
EROWS = EPAD // CH
RPT = EROWS // NTILES
SUP = 20
NSUP = RPT // SUP
NBUF = 4
NQ = SUP // NBUF
ROWS_PT = NPAD // NTILES
ZR = 136
NZ = ROWS_PT // ZR
OPT = B // NTILES


def _body(users_r, pos_r, neg_r, e0_r, rows_r, cols_r, vals_r,
          u_o, p_o, n_o, s0_o, s1_o, s2_o,
          acc, colbuf, rowbuf, valbuf, zero_v, rb, semg, sems):
    c = lax.axis_index("c")
    s = lax.axis_index("s")
    cN = c * NPAD

    z16 = jnp.zeros((16,), jnp.float32)

    def zinit(i, carry):
        zero_v[i, pl.ds(0, 16)] = z16
        zero_v[i, pl.ds(16, 16)] = z16
        return carry
    lax.fori_loop(0, ZR, zinit, 0)

    def do_layer(src_r, dst_r):
        for z in range(NZ):
            pltpu.sync_copy(zero_v, acc.at[pl.ds(s * ROWS_PT + z * ZR, ZR)])
        plsc.subcore_barrier()

        def gissue(b, row):
            pltpu.async_copy(src_r.at[colbuf.at[row]], rb.at[b], semg.at[b])

        def process(b, row):
            pltpu.make_async_copy(
                src_r.at[colbuf.at[row]], rb.at[b], semg.at[b]).wait()

            def qbody(q, carry2):
                for h in range(2):
                    qq = q * 2 + h
                    vv = valbuf[row, pl.ds(qq * 16, 16)]
                    for i in range(16):
                        j = qq * 16 + i
                        v = vv[i]
                        rb[b, j, pl.ds(0, 16)] = rb[b, j, pl.ds(0, 16)] * v
                        rb[b, j, pl.ds(16, 16)] = rb[b, j, pl.ds(16, 16)] * v
                return carry2
            lax.fori_loop(0, CH // 32, qbody, 0)
            pltpu.async_copy(rb.at[b], acc.at[rowbuf.at[row]],
                             sems.at[b], add=True)

        def swait(b, row):
            pltpu.make_async_copy(
                rb.at[b], acc.at[rowbuf.at[row]], sems.at[b]).wait()

        def superchunk(sc, carry):
            row0 = s * RPT + sc * SUP
            pltpu.sync_copy(cols_r.at[pl.ds(row0, SUP)], colbuf)
            pltpu.sync_copy(rows_r.at[pl.ds(row0, SUP)], rowbuf)
            pltpu.sync_copy(vals_r.at[pl.ds(row0, SUP)], valbuf)

            def addrow(k, carry2):
                for q in range(CH // 16):
                    colbuf[k, pl.ds(q * 16, 16)] = (
                        colbuf[k, pl.ds(q * 16, 16)] + cN)
                return carry2
            lax.fori_loop(0, SUP, addrow, 0)

            def quad(k, carry2):
                for b in range(NBUF):
                    gissue(b, k * NBUF + b)
                for b in range(NBUF):
                    process(b, k * NBUF + b)
                for b in range(NBUF):
                    swait(b, k * NBUF + b)
                return carry2
            lax.fori_loop(0, NQ, quad, 0)
            return carry
        lax.fori_loop(0, NSUP, superchunk, 0)
        plsc.subcore_barrier()

        for z in range(NZ):
            r0 = s * ROWS_PT + z * ZR
            pltpu.sync_copy(acc.at[pl.ds(r0, ZR)], dst_r.at[pl.ds(cN + r0, ZR)])
        plsc.subcore_barrier()

    do_layer(e0_r, s0_o)
    do_layer(s0_o, s1_o)
    do_layer(s1_o, s2_o)

    def emit(idx_hbm, off, out_r):
        idx_v = colbuf.at[0]
        gA = zero_v
        gB = rb.at[0]
        for ch in range(OPT // 128):
            r0 = s * OPT + ch * 128
            pltpu.sync_copy(idx_hbm.at[pl.ds(r0, 128)], idx_v)
            offc = off + cN
            for k in range(8):
                idx_v[pl.ds(k * 16, 16)] = idx_v[pl.ds(k * 16, 16)] + offc
            pltpu.sync_copy(e0_r.at[idx_v], gA.at[pl.ds(0, 128)])
            for li, tref in enumerate((s0_o, s1_o, s2_o)):
                pltpu.sync_copy(tref.at[idx_v], gB)
                scale = 0.25 if li == 2 else None

                def sbody(j, carry):
                    for q in (0, 16):
                        a = gA[j, pl.ds(q, 16)]
                        b = gB[j, pl.ds(q, 16)]
                        r = a + b
                        if scale is not None:
                            r = r * scale
                        gA[j, pl.ds(q, 16)] = r
                    return carry
                lax.fori_loop(0, 128, sbody, 0)
            pltpu.sync_copy(gA.at[pl.ds(0, 128)], out_r.at[c, pl.ds(r0, 128)])

    emit(users_r, 0, u_o)
    emit(pos_r, NU, p_o)
    emit(neg_r, NU, n_o)


@functools.partial(jax.jit, static_argnames=())
def kernel(users, pos_items, neg_items, emb_weight, adj_rows, adj_cols, adj_vals):
    users = users.astype(jnp.int32)
    pos_items = pos_items.astype(jnp.int32)
    neg_items = neg_items.astype(jnp.int32)
    npad_e = EPAD - E
    adj_rows = jnp.concatenate(
        [adj_rows.astype(jnp.int32), jnp.full((npad_e,), N, jnp.int32)]
    ).reshape(EROWS, CH)
    adj_cols = jnp.concatenate(
        [adj_cols.astype(jnp.int32), jnp.full((npad_e,), N, jnp.int32)]
    ).reshape(EROWS, CH)
    adj_vals = jnp.concatenate(
        [adj_vals, jnp.zeros((npad_e,), jnp.float32)]
    ).reshape(EROWS, CH)
    e0 = jnp.zeros((2 * NPAD, DH), jnp.float32)
    e0 = e0.at[:N].set(emb_weight[:, :DH]).at[NPAD:NPAD + N].set(emb_weight[:, DH:])

    f32 = jnp.float32
    tbl = jax.ShapeDtypeStruct((2 * NPAD, DH), f32)
    out2 = jax.ShapeDtypeStruct((2, B, DH), f32)

    run = pl.kernel(
        _body,
        out_type=[out2, out2, out2, tbl, tbl, tbl],
        mesh=plsc.VectorSubcoreMesh(core_axis_name="c", subcore_axis_name="s"),
        compiler_params=pltpu.CompilerParams(use_tc_tiling_on_sc=False),
        scratch_types=[
            pltpu.VMEM_SHARED((NPAD, DH), f32),
            pltpu.VMEM((SUP, CH), jnp.int32),
            pltpu.VMEM((SUP, CH), jnp.int32),
            pltpu.VMEM((SUP, CH), f32),
            pltpu.VMEM((ZR, DH), f32),
            pltpu.VMEM((NBUF, CH, DH), f32),
            pltpu.SemaphoreType.DMA((NBUF,)),
            pltpu.SemaphoreType.DMA((NBUF,)),
        ],
    )
    u2, p2, n2, _, _, _ = run(users, pos_items, neg_items, e0,
                              adj_rows, adj_cols, adj_vals)
    u_e = jnp.concatenate([u2[0], u2[1]], axis=1)
    pos_e = jnp.concatenate([p2[0], p2[1]], axis=1)
    neg_e = jnp.concatenate([n2[0], n2[1]], axis=1)
    return (u_e, pos_e, neg_e)


# speedup vs baseline: 1.4938x; 1.4938x over previous
import functools
import jax
import jax.numpy as jnp
from jax import lax
from jax.experimental import pallas as pl
from jax.experimental.pallas import tpu as pltpu
from jax.experimental.pallas import tpu_sc as plsc

NU = 25000
NI = 25000
N = NU + NI
NPAD = 50048
E = 800000
EPAD = 819200
D = 64
DH = 32
B = 4096
NTILES = 16
CH = 80
EROWS = EPAD // CH
RPT = EROWS // NTILES
SUP = 32
NSUP = RPT // SUP
NBUF = 4
NQ = SUP // NBUF
ROWS_PT = NPAD // NTILES
ZR = 136
NZ = ROWS_PT // ZR
OPT = B // NTILES


def _body(users_r, pos_r, neg_r, e0_r, rows_r, cols_r, vals_r,
          u_o, p_o, n_o, s0_o, s1_o, s2_o,
          acc, colbuf, rowbuf, valbuf, zero_v, rb, idx_v, gb_v, semg, sems):
    c = lax.axis_index("c")
    s = lax.axis_index("s")
    cN = c * NPAD

    z16 = jnp.zeros((16,), jnp.float32)

    def zinit(i, carry):
        zero_v[i, pl.ds(0, 16)] = z16
        zero_v[i, pl.ds(16, 16)] = z16
        return carry
    lax.fori_loop(0, ZR, zinit, 0)

    def do_layer(src_r, dst_r):
        for z in range(NZ):
            pltpu.sync_copy(zero_v, acc.at[pl.ds(s * ROWS_PT + z * ZR, ZR)])
        plsc.subcore_barrier()

        def gissue(b, row):
            pltpu.async_copy(src_r.at[colbuf.at[row]], rb.at[b], semg.at[b])

        def process(b, row):
            pltpu.make_async_copy(
                src_r.at[colbuf.at[row]], rb.at[b], semg.at[b]).wait()

            def qbody(q, carry2):
                vv = valbuf[row, pl.ds(q * 16, 16)]
                for i in range(16):
                    j = q * 16 + i
                    v = vv[i]
                    rb[b, j, pl.ds(0, 16)] = rb[b, j, pl.ds(0, 16)] * v
                    rb[b, j, pl.ds(16, 16)] = rb[b, j, pl.ds(16, 16)] * v
                return carry2
            lax.fori_loop(0, CH // 16, qbody, 0)
            pltpu.async_copy(rb.at[b], acc.at[rowbuf.at[row]],
                             sems.at[b], add=True)

        def swait(b, row):
            pltpu.make_async_copy(
                rb.at[b], acc.at[rowbuf.at[row]], sems.at[b]).wait()

        def superchunk(sc, carry):
            row0 = s * RPT + sc * SUP
            pltpu.sync_copy(cols_r.at[pl.ds(row0, SUP)], colbuf)
            pltpu.sync_copy(rows_r.at[pl.ds(row0, SUP)], rowbuf)
            pltpu.sync_copy(vals_r.at[pl.ds(row0, SUP)], valbuf)

            def addrow(k, carry2):
                for q in range(CH // 16):
                    colbuf[k, pl.ds(q * 16, 16)] = (
                        colbuf[k, pl.ds(q * 16, 16)] + cN)
                return carry2
            lax.fori_loop(0, SUP, addrow, 0)

            def quad(k, carry2):
                for b in range(NBUF):
                    gissue(b, k * NBUF + b)
                for b in range(NBUF):
                    process(b, k * NBUF + b)
                for b in range(NBUF):
                    swait(b, k * NBUF + b)
                return carry2
            lax.fori_loop(0, NQ, quad, 0)
            return carry
        lax.fori_loop(0, NSUP, superchunk, 0)
        plsc.subcore_barrier()

        for z in range(NZ):
            r0 = s * ROWS_PT + z * ZR
            pltpu.sync_copy(acc.at[pl.ds(r0, ZR)], dst_r.at[pl.ds(cN + r0, ZR)])
        plsc.subcore_barrier()

    do_layer(e0_r, s0_o)
    do_layer(s0_o, s1_o)
    do_layer(s1_o, s2_o)

    def emit(idx_hbm, off, out_r):
        gA = zero_v
        gB = gb_v
        for ch in range(OPT // 128):
            r0 = s * OPT + ch * 128
            pltpu.sync_copy(idx_hbm.at[pl.ds(r0, 128)], idx_v)
            offc = off + cN
            for k in range(8):
                idx_v[pl.ds(k * 16, 16)] = idx_v[pl.ds(k * 16, 16)] + offc
            pltpu.sync_copy(e0_r.at[idx_v], gA.at[pl.ds(0, 128)])
            for li, tref in enumerate((s0_o, s1_o, s2_o)):
                pltpu.sync_copy(tref.at[idx_v], gB)
                scale = 0.25 if li == 2 else None

                def sbody(j, carry):
                    for q in (0, 16):
                        a = gA[j, pl.ds(q, 16)]
                        b = gB[j, pl.ds(q, 16)]
                        r = a + b
                        if scale is not None:
                            r = r * scale
                        gA[j, pl.ds(q, 16)] = r
                    return carry
                lax.fori_loop(0, 128, sbody, 0)
            pltpu.sync_copy(gA.at[pl.ds(0, 128)], out_r.at[c, pl.ds(r0, 128)])

    emit(users_r, 0, u_o)
    emit(pos_r, NU, p_o)
    emit(neg_r, NU, n_o)


@functools.partial(jax.jit, static_argnames=())
def kernel(users, pos_items, neg_items, emb_weight, adj_rows, adj_cols, adj_vals):
    users = users.astype(jnp.int32)
    pos_items = pos_items.astype(jnp.int32)
    neg_items = neg_items.astype(jnp.int32)
    npad_e = EPAD - E
    adj_rows = jnp.concatenate(
        [adj_rows.astype(jnp.int32), jnp.full((npad_e,), N, jnp.int32)]
    ).reshape(EROWS, CH)
    adj_cols = jnp.concatenate(
        [adj_cols.astype(jnp.int32), jnp.full((npad_e,), N, jnp.int32)]
    ).reshape(EROWS, CH)
    adj_vals = jnp.concatenate(
        [adj_vals, jnp.zeros((npad_e,), jnp.float32)]
    ).reshape(EROWS, CH)
    e0 = jnp.zeros((2 * NPAD, DH), jnp.float32)
    e0 = e0.at[:N].set(emb_weight[:, :DH]).at[NPAD:NPAD + N].set(emb_weight[:, DH:])

    f32 = jnp.float32
    tbl = jax.ShapeDtypeStruct((2 * NPAD, DH), f32)
    out2 = jax.ShapeDtypeStruct((2, B, DH), f32)

    run = pl.kernel(
        _body,
        out_type=[out2, out2, out2, tbl, tbl, tbl],
        mesh=plsc.VectorSubcoreMesh(core_axis_name="c", subcore_axis_name="s"),
        compiler_params=pltpu.CompilerParams(use_tc_tiling_on_sc=False),
        scratch_types=[
            pltpu.VMEM_SHARED((NPAD, DH), f32),
            pltpu.VMEM((SUP, CH), jnp.int32),
            pltpu.VMEM((SUP, CH), jnp.int32),
            pltpu.VMEM((SUP, CH), f32),
            pltpu.VMEM((ZR, DH), f32),
            pltpu.VMEM((NBUF, CH, DH), f32),
            pltpu.VMEM((128,), jnp.int32),
            pltpu.VMEM((128, DH), f32),
            pltpu.SemaphoreType.DMA((NBUF,)),
            pltpu.SemaphoreType.DMA((NBUF,)),
        ],
    )
    u2, p2, n2, _, _, _ = run(users, pos_items, neg_items, e0,
                              adj_rows, adj_cols, adj_vals)
    u_e = jnp.concatenate([u2[0], u2[1]], axis=1)
    pos_e = jnp.concatenate([p2[0], p2[1]], axis=1)
    neg_e = jnp.concatenate([n2[0], n2[1]], axis=1)
    return (u_e, pos_e, neg_e)


# speedup vs baseline: 2.4153x; 1.6169x over previous
import functools
import jax
import jax.numpy as jnp
from jax import lax
from jax.experimental import pallas as pl
from jax.experimental.pallas import tpu as pltpu
from jax.experimental.pallas import tpu_sc as plsc

NU = 25000
NI = 25000
N = NU + NI
NPAD = 50048
E = 800000
EPAD = 819200
D = 64
DH = 32
B = 4096
NTILES = 16
CH = 80
EROWS = EPAD // CH
RPT = EROWS // NTILES
SUP = 32
NSUP = RPT // SUP
NBUF = 4
NQ = SUP // NBUF
ROWS_PT = NPAD // NTILES
ZR = 136
NZ = ROWS_PT // ZR
OPT = B // NTILES


def _body(users_r, pos_r, neg_r, e0_r, rows_r, cols_r, vals_r,
          u_o, p_o, n_o, s0_o, s1_o, s2_o,
          acc, colbuf, rowbuf, valbuf, zero_v, rb, idx_v, gb_v, semg, sems):
    c = lax.axis_index("c")
    s = lax.axis_index("s")
    cN = c * NPAD

    z16 = jnp.zeros((16,), jnp.float32)

    def zinit(i, carry):
        zero_v[i, pl.ds(0, 16)] = z16
        zero_v[i, pl.ds(16, 16)] = z16
        return carry
    lax.fori_loop(0, ZR, zinit, 0)

    def do_layer(src_r, dst_r):
        for z in range(NZ):
            pltpu.sync_copy(zero_v, acc.at[pl.ds(s * ROWS_PT + z * ZR, ZR)])
        plsc.subcore_barrier()

        def gissue(b, row):
            pltpu.async_copy(src_r.at[colbuf.at[row]], rb.at[b], semg.at[b])

        def process(b, row):
            pltpu.make_async_copy(
                src_r.at[colbuf.at[row]], rb.at[b], semg.at[b]).wait()

            def qbody(q, carry2):
                vv = valbuf[row, pl.ds(q * 16, 16)]
                for i in range(16):
                    j = q * 16 + i
                    v = vv[i]
                    rb[b, j, pl.ds(0, 16)] = rb[b, j, pl.ds(0, 16)] * v
                    rb[b, j, pl.ds(16, 16)] = rb[b, j, pl.ds(16, 16)] * v
                return carry2
            lax.fori_loop(0, CH // 16, qbody, 0)
            pltpu.async_copy(rb.at[b], acc.at[rowbuf.at[row]],
                             sems.at[b], add=True)

        def swait(b, row):
            pltpu.make_async_copy(
                rb.at[b], acc.at[rowbuf.at[row]], sems.at[b]).wait()

        def superchunk(sc, carry):
            row0 = s * RPT + sc * SUP
            pltpu.sync_copy(cols_r.at[pl.ds(row0, SUP)], colbuf)
            pltpu.sync_copy(rows_r.at[pl.ds(row0, SUP)], rowbuf)
            pltpu.sync_copy(vals_r.at[pl.ds(row0, SUP)], valbuf)

            def addrow(k, carry2):
                for q in range(CH // 16):
                    colbuf[k, pl.ds(q * 16, 16)] = (
                        colbuf[k, pl.ds(q * 16, 16)] + cN)
                return carry2
            lax.fori_loop(0, SUP, addrow, 0)

            def quad(k, carry2):
                for b in range(NBUF):
                    gissue(b, k * NBUF + b)
                for b in range(NBUF):
                    process(b, k * NBUF + b)
                for b in range(NBUF):
                    swait(b, k * NBUF + b)
                return carry2
            lax.fori_loop(0, NQ, quad, 0)
            return carry
        lax.fori_loop(0, NSUP, superchunk, 0)
        plsc.subcore_barrier()

        for z in range(NZ):
            r0 = s * ROWS_PT + z * ZR
            pltpu.sync_copy(acc.at[pl.ds(r0, ZR)], dst_r.at[pl.ds(cN + r0, ZR)])
        plsc.subcore_barrier()

    do_layer(e0_r, s0_o)
    do_layer(s0_o, s1_o)
    do_layer(s1_o, s2_o)

    def emit(idx_hbm, off, out_r):
        gA = zero_v
        gB = gb_v
        for ch in range(OPT // 128):
            r0 = s * OPT + ch * 128
            pltpu.sync_copy(idx_hbm.at[pl.ds(r0, 128)], idx_v)
            offc = off + cN
            for k in range(8):
                idx_v[pl.ds(k * 16, 16)] = idx_v[pl.ds(k * 16, 16)] + offc
            pltpu.sync_copy(e0_r.at[idx_v], gA.at[pl.ds(0, 128)])
            for li, tref in enumerate((s0_o, s1_o, s2_o)):
                pltpu.sync_copy(tref.at[idx_v], gB)
                scale = 0.25 if li == 2 else None

                def sbody(j, carry):
                    for q in (0, 16):
                        a = gA[j, pl.ds(q, 16)]
                        b = gB[j, pl.ds(q, 16)]
                        r = a + b
                        if scale is not None:
                            r = r * scale
                        gA[j, pl.ds(q, 16)] = r
                    return carry
                lax.fori_loop(0, 128, sbody, 0)
            pltpu.sync_copy(gA.at[pl.ds(0, 128)], out_r.at[c, pl.ds(r0, 128)])

    emit(users_r, 0, u_o)
    emit(pos_r, NU, p_o)
    emit(neg_r, NU, n_o)


@functools.partial(jax.jit, static_argnames=())
def kernel(users, pos_items, neg_items, emb_weight, adj_rows, adj_cols, adj_vals):
    users = users.astype(jnp.int32)
    pos_items = pos_items.astype(jnp.int32)
    neg_items = neg_items.astype(jnp.int32)
    npad_e = EPAD - E
    spread = (jnp.arange(npad_e, dtype=jnp.int32) * 131) % N
    adj_rows = jnp.concatenate(
        [adj_rows.astype(jnp.int32), spread]).reshape(EROWS, CH)
    adj_cols = jnp.concatenate(
        [adj_cols.astype(jnp.int32), spread]).reshape(EROWS, CH)
    adj_vals = jnp.concatenate(
        [adj_vals, jnp.zeros((npad_e,), jnp.float32)]).reshape(EROWS, CH)
    e0 = jnp.zeros((2 * NPAD, DH), jnp.float32)
    e0 = e0.at[:N].set(emb_weight[:, :DH]).at[NPAD:NPAD + N].set(emb_weight[:, DH:])

    f32 = jnp.float32
    tbl = jax.ShapeDtypeStruct((2 * NPAD, DH), f32)
    out2 = jax.ShapeDtypeStruct((2, B, DH), f32)

    run = pl.kernel(
        _body,
        out_type=[out2, out2, out2, tbl, tbl, tbl],
        mesh=plsc.VectorSubcoreMesh(core_axis_name="c", subcore_axis_name="s"),
        compiler_params=pltpu.CompilerParams(use_tc_tiling_on_sc=False),
        scratch_types=[
            pltpu.VMEM_SHARED((NPAD, DH), f32),
            pltpu.VMEM((SUP, CH), jnp.int32),
            pltpu.VMEM((SUP, CH), jnp.int32),
            pltpu.VMEM((SUP, CH), f32),
            pltpu.VMEM((ZR, DH), f32),
            pltpu.VMEM((NBUF, CH, DH), f32),
            pltpu.VMEM((128,), jnp.int32),
            pltpu.VMEM((128, DH), f32),
            pltpu.SemaphoreType.DMA((NBUF,)),
            pltpu.SemaphoreType.DMA((NBUF,)),
        ],
    )
    u2, p2, n2, _, _, _ = run(users, pos_items, neg_items, e0,
                              adj_rows, adj_cols, adj_vals)
    u_e = jnp.concatenate([u2[0], u2[1]], axis=1)
    pos_e = jnp.concatenate([p2[0], p2[1]], axis=1)
    neg_e = jnp.concatenate([n2[0], n2[1]], axis=1)
    return (u_e, pos_e, neg_e)


# speedup vs baseline: 2.7072x; 1.1209x over previous
import functools
import jax
import jax.numpy as jnp
from jax import lax
from jax.experimental import pallas as pl
from jax.experimental.pallas import tpu as pltpu
from jax.experimental.pallas import tpu_sc as plsc

NU = 25000
NI = 25000
N = NU + NI
NPAD = 50048
E = 800000
EPAD = 819200
D = 64
DH = 32
B = 4096
NTILES = 16
CH = 80
EROWS = EPAD // CH
RPT = EROWS // NTILES
SUP = 32
NSUP = RPT // SUP
NBUF = 4
NQ = SUP // NBUF
ROWS_PT = NPAD // NTILES
ZR = 136
NZ = ROWS_PT // ZR
OPT = B // NTILES


def _body(users_r, pos_r, neg_r, e0_r, rows_r, cols_r, vals_r,
          u_o, p_o, n_o, s0_o, s1_o, s2_o,
          acc, colbuf, rowbuf, valbuf, zero_v, rb, idx_v, gb_v, semg, sems):
    c = lax.axis_index("c")
    s = lax.axis_index("s")
    cN = c * NPAD

    z32 = jnp.zeros((32,), jnp.bfloat16)

    def zinit(i, carry):
        zero_v[i, pl.ds(0, 32)] = z32
        return carry
    lax.fori_loop(0, ZR, zinit, 0)

    def do_layer(src_r, dst_r):
        for z in range(NZ):
            pltpu.sync_copy(zero_v, acc.at[pl.ds(s * ROWS_PT + z * ZR, ZR)])
        plsc.subcore_barrier()

        def gissue(b, row):
            pltpu.async_copy(src_r.at[colbuf.at[row]], rb.at[b], semg.at[b])

        def process(b, row):
            pltpu.make_async_copy(
                src_r.at[colbuf.at[row]], rb.at[b], semg.at[b]).wait()

            def qbody(q, carry2):
                vv = valbuf[row, pl.ds(q * 16, 16)]
                for i in range(16):
                    j = q * 16 + i
                    v = vv[i]
                    ab = rb[b, j, pl.ds(0, 32)]
                    lo, hi = plsc.unpack(ab, format=plsc.PackFormat.INTERLEAVED)
                    rb[b, j, pl.ds(0, 32)] = plsc.pack(
                        lo * v, hi * v, format=plsc.PackFormat.INTERLEAVED)
                return carry2
            lax.fori_loop(0, CH // 16, qbody, 0)
            pltpu.async_copy(rb.at[b], acc.at[rowbuf.at[row]],
                             sems.at[b], add=True)

        def swait(b, row):
            pltpu.make_async_copy(
                rb.at[b], acc.at[rowbuf.at[row]], sems.at[b]).wait()

        def superchunk(sc, carry):
            row0 = s * RPT + sc * SUP
            pltpu.sync_copy(cols_r.at[pl.ds(row0, SUP)], colbuf)
            pltpu.sync_copy(rows_r.at[pl.ds(row0, SUP)], rowbuf)
            pltpu.sync_copy(vals_r.at[pl.ds(row0, SUP)], valbuf)

            def addrow(k, carry2):
                for q in range(CH // 16):
                    colbuf[k, pl.ds(q * 16, 16)] = (
                        colbuf[k, pl.ds(q * 16, 16)] + cN)
                return carry2
            lax.fori_loop(0, SUP, addrow, 0)

            def quad(k, carry2):
                for b in range(NBUF):
                    gissue(b, k * NBUF + b)
                for b in range(NBUF):
                    process(b, k * NBUF + b)
                for b in range(NBUF):
                    swait(b, k * NBUF + b)
                return carry2
            lax.fori_loop(0, NQ, quad, 0)
            return carry
        lax.fori_loop(0, NSUP, superchunk, 0)
        plsc.subcore_barrier()

        for z in range(NZ):
            r0 = s * ROWS_PT + z * ZR
            pltpu.sync_copy(acc.at[pl.ds(r0, ZR)], dst_r.at[pl.ds(cN + r0, ZR)])
        plsc.subcore_barrier()

    do_layer(e0_r, s0_o)
    do_layer(s0_o, s1_o)
    do_layer(s1_o, s2_o)

    def emit(idx_hbm, off, out_r):
        gA = zero_v
        gB = gb_v
        for ch in range(OPT // 128):
            r0 = s * OPT + ch * 128
            pltpu.sync_copy(idx_hbm.at[pl.ds(r0, 128)], idx_v)
            offc = off + cN
            for k in range(8):
                idx_v[pl.ds(k * 16, 16)] = idx_v[pl.ds(k * 16, 16)] + offc
            pltpu.sync_copy(e0_r.at[idx_v], gA.at[pl.ds(0, 128)])
            for li, tref in enumerate((s0_o, s1_o, s2_o)):
                pltpu.sync_copy(tref.at[idx_v], gB)
                scale = 0.25 if li == 2 else None

                def sbody(j, carry):
                    a = gA[j, pl.ds(0, 32)]
                    b = gB[j, pl.ds(0, 32)]
                    r = a + b
                    if scale is not None:
                        r = r * jnp.bfloat16(0.25)
                    gA[j, pl.ds(0, 32)] = r
                    return carry
                lax.fori_loop(0, 128, sbody, 0)
            pltpu.sync_copy(gA.at[pl.ds(0, 128)], out_r.at[c, pl.ds(r0, 128)])

    emit(users_r, 0, u_o)
    emit(pos_r, NU, p_o)
    emit(neg_r, NU, n_o)


@functools.partial(jax.jit, static_argnames=())
def kernel(users, pos_items, neg_items, emb_weight, adj_rows, adj_cols, adj_vals):
    users = users.astype(jnp.int32)
    pos_items = pos_items.astype(jnp.int32)
    neg_items = neg_items.astype(jnp.int32)
    npad_e = EPAD - E
    spread = (jnp.arange(npad_e, dtype=jnp.int32) * 131) % N
    adj_rows = jnp.concatenate(
        [adj_rows.astype(jnp.int32), spread]).reshape(EROWS, CH)
    adj_cols = jnp.concatenate(
        [adj_cols.astype(jnp.int32), spread]).reshape(EROWS, CH)
    adj_vals = jnp.concatenate(
        [adj_vals, jnp.zeros((npad_e,), jnp.float32)]).reshape(EROWS, CH)
    bf16 = jnp.bfloat16
    emb_bf = emb_weight.astype(bf16)
    e0 = jnp.zeros((2 * NPAD, DH), bf16)
    e0 = e0.at[:N].set(emb_bf[:, :DH]).at[NPAD:NPAD + N].set(emb_bf[:, DH:])

    f32 = jnp.float32
    tbl = jax.ShapeDtypeStruct((2 * NPAD, DH), bf16)
    out2 = jax.ShapeDtypeStruct((2, B, DH), bf16)

    run = pl.kernel(
        _body,
        out_type=[out2, out2, out2, tbl, tbl, tbl],
        mesh=plsc.VectorSubcoreMesh(core_axis_name="c", subcore_axis_name="s"),
        compiler_params=pltpu.CompilerParams(use_tc_tiling_on_sc=False, needs_layout_passes=False),
        scratch_types=[
            pltpu.VMEM_SHARED((NPAD, DH), jnp.bfloat16),
            pltpu.VMEM((SUP, CH), jnp.int32),
            pltpu.VMEM((SUP, CH), jnp.int32),
            pltpu.VMEM((SUP, CH), f32),
            pltpu.VMEM((ZR, DH), jnp.bfloat16),
            pltpu.VMEM((NBUF, CH, DH), jnp.bfloat16),
            pltpu.VMEM((128,), jnp.int32),
            pltpu.VMEM((128, DH), jnp.bfloat16),
            pltpu.SemaphoreType.DMA((NBUF,)),
            pltpu.SemaphoreType.DMA((NBUF,)),
        ],
    )
    u2, p2, n2, _, _, _ = run(users, pos_items, neg_items, e0,
                              adj_rows, adj_cols, adj_vals)
    u_e = jnp.concatenate([u2[0], u2[1]], axis=1).astype(f32)
    pos_e = jnp.concatenate([p2[0], p2[1]], axis=1).astype(f32)
    neg_e = jnp.concatenate([n2[0], n2[1]], axis=1).astype(f32)
    return (u_e, pos_e, neg_e)


# speedup vs baseline: 3.3762x; 1.2471x over previous
import functools
import jax
import jax.numpy as jnp
from jax import lax
from jax.experimental import pallas as pl
from jax.experimental.pallas import tpu as pltpu
from jax.experimental.pallas import tpu_sc as plsc

NU = 25000
NI = 25000
N = NU + NI
NPAD = 50048
E = 800000
EPAD = 819200
D = 64
DH = 32
B = 4096
NTILES = 16
CH = 80
EROWS = EPAD // CH
RPT = EROWS // NTILES
SUP = 32
NSUP = RPT // SUP
NBUF = 8
NQ = SUP // NBUF
ROWS_PT = NPAD // NTILES
ZR = 136
NZ = ROWS_PT // ZR
OPT = B // NTILES


def _body(users_r, pos_r, neg_r, e0_r, rows_r, cols_r, vals_r,
          u_o, p_o, n_o, s0_o, s1_o, s2_o,
          acc, colbuf, rowbuf, valbuf, zero_v, rb, idx_v, gb_v, semg, sems):
    c = lax.axis_index("c")
    s = lax.axis_index("s")
    cN = c * NPAD

    z32 = jnp.zeros((32,), jnp.bfloat16)

    def zinit(i, carry):
        zero_v[i, pl.ds(0, 32)] = z32
        return carry
    lax.fori_loop(0, ZR, zinit, 0)

    def do_layer(src_r, dst_r):
        for z in range(NZ):
            pltpu.sync_copy(zero_v, acc.at[pl.ds(s * ROWS_PT + z * ZR, ZR)])
        plsc.subcore_barrier()

        def gissue(b, row):
            pltpu.async_copy(src_r.at[colbuf.at[row]], rb.at[b], semg.at[b])

        def process(b, row):
            pltpu.make_async_copy(
                src_r.at[colbuf.at[row]], rb.at[b], semg.at[b]).wait()

            def qbody(q, carry2):
                vv = valbuf[row, pl.ds(q * 16, 16)]
                for i in range(16):
                    j = q * 16 + i
                    v = vv[i]
                    ab = rb[b, j, pl.ds(0, 32)]
                    lo, hi = plsc.unpack(ab, format=plsc.PackFormat.INTERLEAVED)
                    rb[b, j, pl.ds(0, 32)] = plsc.pack(
                        lo * v, hi * v, format=plsc.PackFormat.INTERLEAVED)
                return carry2
            lax.fori_loop(0, CH // 16, qbody, 0)
            pltpu.async_copy(rb.at[b], acc.at[rowbuf.at[row]],
                             sems.at[b], add=True)

        def swait(b, row):
            pltpu.make_async_copy(
                rb.at[b], acc.at[rowbuf.at[row]], sems.at[b]).wait()

        def superchunk(sc, carry):
            row0 = s * RPT + sc * SUP
            pltpu.sync_copy(cols_r.at[pl.ds(row0, SUP)], colbuf)
            pltpu.sync_copy(rows_r.at[pl.ds(row0, SUP)], rowbuf)
            pltpu.sync_copy(vals_r.at[pl.ds(row0, SUP)], valbuf)

            def addrow(k, carry2):
                for q in range(CH // 16):
                    colbuf[k, pl.ds(q * 16, 16)] = (
                        colbuf[k, pl.ds(q * 16, 16)] + cN)
                return carry2
            lax.fori_loop(0, SUP, addrow, 0)

            def quad(k, carry2):
                for b in range(NBUF):
                    gissue(b, k * NBUF + b)
                for b in range(NBUF):
                    process(b, k * NBUF + b)
                for b in range(NBUF):
                    swait(b, k * NBUF + b)
                return carry2
            lax.fori_loop(0, NQ, quad, 0)
            return carry
        lax.fori_loop(0, NSUP, superchunk, 0)
        plsc.subcore_barrier()

        for z in range(NZ):
            r0 = s * ROWS_PT + z * ZR
            pltpu.sync_copy(acc.at[pl.ds(r0, ZR)], dst_r.at[pl.ds(cN + r0, ZR)])
        plsc.subcore_barrier()

    do_layer(e0_r, s0_o)
    do_layer(s0_o, s1_o)
    do_layer(s1_o, s2_o)

    def emit(idx_hbm, off, out_r):
        gA = zero_v
        gB = gb_v
        for ch in range(OPT // 128):
            r0 = s * OPT + ch * 128
            pltpu.sync_copy(idx_hbm.at[pl.ds(r0, 128)], idx_v)
            offc = off + cN
            for k in range(8):
                idx_v[pl.ds(k * 16, 16)] = idx_v[pl.ds(k * 16, 16)] + offc
            pltpu.sync_copy(e0_r.at[idx_v], gA.at[pl.ds(0, 128)])
            for li, tref in enumerate((s0_o, s1_o, s2_o)):
                pltpu.sync_copy(tref.at[idx_v], gB)
                scale = 0.25 if li == 2 else None

                def sbody(j, carry):
                    a = gA[j, pl.ds(0, 32)]
                    b = gB[j, pl.ds(0, 32)]
                    r = a + b
                    if scale is not None:
                        r = r * jnp.bfloat16(0.25)
                    gA[j, pl.ds(0, 32)] = r
                    return carry
                lax.fori_loop(0, 128, sbody, 0)
            pltpu.sync_copy(gA.at[pl.ds(0, 128)], out_r.at[c, pl.ds(r0, 128)])

    emit(users_r, 0, u_o)
    emit(pos_r, NU, p_o)
    emit(neg_r, NU, n_o)


@functools.partial(jax.jit, static_argnames=())
def kernel(users, pos_items, neg_items, emb_weight, adj_rows, adj_cols, adj_vals):
    users = users.astype(jnp.int32)
    pos_items = pos_items.astype(jnp.int32)
    neg_items = neg_items.astype(jnp.int32)
    npad_e = EPAD - E
    spread = (jnp.arange(npad_e, dtype=jnp.int32) * 131) % N
    adj_rows = jnp.concatenate(
        [adj_rows.astype(jnp.int32), spread]).reshape(EROWS, CH)
    adj_cols = jnp.concatenate(
        [adj_cols.astype(jnp.int32), spread]).reshape(EROWS, CH)
    adj_vals = jnp.concatenate(
        [adj_vals, jnp.zeros((npad_e,), jnp.float32)]).reshape(EROWS, CH)
    bf16 = jnp.bfloat16
    emb_bf = emb_weight.astype(bf16)
    e0 = jnp.zeros((2 * NPAD, DH), bf16)
    e0 = e0.at[:N].set(emb_bf[:, :DH]).at[NPAD:NPAD + N].set(emb_bf[:, DH:])

    f32 = jnp.float32
    tbl = jax.ShapeDtypeStruct((2 * NPAD, DH), bf16)
    out2 = jax.ShapeDtypeStruct((2, B, DH), bf16)

    run = pl.kernel(
        _body,
        out_type=[out2, out2, out2, tbl, tbl, tbl],
        mesh=plsc.VectorSubcoreMesh(core_axis_name="c", subcore_axis_name="s"),
        compiler_params=pltpu.CompilerParams(use_tc_tiling_on_sc=False, needs_layout_passes=False),
        scratch_types=[
            pltpu.VMEM_SHARED((NPAD, DH), jnp.bfloat16),
            pltpu.VMEM((SUP, CH), jnp.int32),
            pltpu.VMEM((SUP, CH), jnp.int32),
            pltpu.VMEM((SUP, CH), f32),
            pltpu.VMEM((ZR, DH), jnp.bfloat16),
            pltpu.VMEM((NBUF, CH, DH), jnp.bfloat16),
            pltpu.VMEM((128,), jnp.int32),
            pltpu.VMEM((128, DH), jnp.bfloat16),
            pltpu.SemaphoreType.DMA((NBUF,)),
            pltpu.SemaphoreType.DMA((NBUF,)),
        ],
    )
    u2, p2, n2, _, _, _ = run(users, pos_items, neg_items, e0,
                              adj_rows, adj_cols, adj_vals)
    u_e = jnp.concatenate([u2[0], u2[1]], axis=1).astype(f32)
    pos_e = jnp.concatenate([p2[0], p2[1]], axis=1).astype(f32)
    neg_e = jnp.concatenate([n2[0], n2[1]], axis=1).astype(f32)
    return (u_e, pos_e, neg_e)
